# baseline (speedup 1.0000x reference)
import functools

import jax
import jax.numpy as jnp
from jax import lax
from jax.experimental import pallas as pl
from jax.experimental.pallas import tpu as pltpu
from jax.experimental.pallas import tpu_sc as plsc

VOCAB = 1000000
EMBED_DIMS = 32
BATCH = 16384
EPS = 1e-05

_GROUP = 128 // EMBED_DIMS
_SGRP = 1 << 18
_NGRP = _SGRP
_NC = 2
_NS = 16
_NW = _NC * _NS
_BPW = BATCH // _NW
_CHUNK = 128
_NCHUNK = _BPW // _CHUNK

_VWIN = 8192
_NBLK = _SGRP // _VWIN


def _tc_pack_kernel(t0_ref, t1_ref, t2_ref, t3_ref, o_ref):
    x = jnp.concatenate(
        [t0_ref[...], t1_ref[...], t2_ref[...], t3_ref[...]], axis=0)
    o_ref[...] = x.T


@jax.jit
def _tc_pack(tableT):
    last_blk = VOCAB // _VWIN
    specs = [
        pl.BlockSpec(
            (EMBED_DIMS, _VWIN),
            functools.partial(
                lambda q, i: (0, jnp.minimum(q * _NBLK + i, last_blk)), q))
        for q in range(_GROUP)
    ]
    return pl.pallas_call(
        _tc_pack_kernel,
        grid=(_NBLK,),
        in_specs=specs,
        out_specs=pl.BlockSpec((_VWIN, 128), lambda i: (i, 0)),
        out_shape=jax.ShapeDtypeStruct((_NGRP, 128), jnp.float32),
    )(tableT, tableT, tableT, tableT)


def _sc_gather_kernel(table_hbm, left_hbm, right_hbm, u_hbm, v_hbm,
                      idx_v, rows_v, sem):
    wid = lax.axis_index("s") * _NC + lax.axis_index("c")
    base = wid * _BPW

    for src, dst in ((left_hbm, u_hbm), (right_hbm, v_hbm)):
        pltpu.sync_copy(src.at[wid], idx_v)
        copies = []
        for j in range(_NCHUNK):
            sl = pl.ds(j * _CHUNK, _CHUNK)
            copies.append(
                pltpu.async_copy(table_hbm.at[idx_v.at[j]], rows_v.at[sl], sem))
        for c in copies:
            c.wait()
        pltpu.sync_copy(rows_v, dst.at[pl.ds(base, _BPW)])


@jax.jit
def _sc_gather(table128, left3, right3):
    mesh = plsc.VectorSubcoreMesh(core_axis_name="c", subcore_axis_name="s")
    kfn = functools.partial(
        pl.kernel,
        mesh=mesh,
        out_type=[jax.ShapeDtypeStruct((BATCH, 128), jnp.float32),
                  jax.ShapeDtypeStruct((BATCH, 128), jnp.float32)],
        scratch_types=[
            pltpu.VMEM((_NCHUNK, _CHUNK), jnp.int32),
            pltpu.VMEM((_BPW, 128), jnp.float32),
            pltpu.SemaphoreType.DMA,
        ],
    )(_sc_gather_kernel)
    return kfn(table128, left3, right3)


_TCB = 4096


def _tc_dist_kernel(u_ref, v_ref, lq_ref, rq_ref, o_ref):
    lq = lq_ref[...]
    rq = rq_ref[...]
    lane_q = jax.lax.broadcasted_iota(jnp.int32, (_TCB, 128), 1) // EMBED_DIMS
    u = u_ref[...] * (lane_q == lq[:, None]).astype(jnp.float32)
    v = v_ref[...] * (lane_q == rq[:, None]).astype(jnp.float32)
    uu = jnp.sum(u * u, axis=-1)
    vv = jnp.sum(v * v, axis=-1)
    shift = (lq - rq) & 3
    uv = jnp.zeros((_TCB,), jnp.float32)
    for j in range(_GROUP):
        vr = v if j == 0 else jnp.roll(v, j * EMBED_DIMS, axis=1)
        uvj = jnp.sum(u * vr, axis=-1)
        uv = uv + jnp.where(shift == j, uvj, 0.0)
    alpha = 1.0 - uu
    alpha = jnp.where(alpha <= 0.0, EPS, alpha)
    beta = 1.0 - vv
    beta = jnp.where(beta <= 0.0, EPS, beta)
    gamma = 1.0 + 2.0 * (uu - 2.0 * uv + vv) / alpha / beta
    gamma = jnp.where(gamma < 1.0, 1.0, gamma)
    o_ref[...] = jnp.log(gamma + jnp.sqrt(gamma * gamma - 1.0))


@jax.jit
def _tc_dist(u_rows, v_rows, lq, rq):
    nblk = BATCH // _TCB
    return pl.pallas_call(
        _tc_dist_kernel,
        grid=(nblk,),
        in_specs=[
            pl.BlockSpec((_TCB, 128), lambda i: (i, 0)),
            pl.BlockSpec((_TCB, 128), lambda i: (i, 0)),
            pl.BlockSpec((_TCB,), lambda i: (i,)),
            pl.BlockSpec((_TCB,), lambda i: (i,)),
        ],
        out_specs=pl.BlockSpec((_TCB,), lambda i: (i,)),
        out_shape=jax.ShapeDtypeStruct((BATCH,), jnp.float32),
    )(u_rows, v_rows, lq, rq)


def kernel(left_idx, right_idx, table):
    li = left_idx.astype(jnp.int32)
    ri = right_idx.astype(jnp.int32)
    table128 = _tc_pack(jnp.swapaxes(table, 0, 1))
    left3 = (li & (_SGRP - 1)).reshape(_NW, _NCHUNK, _CHUNK)
    right3 = (ri & (_SGRP - 1)).reshape(_NW, _NCHUNK, _CHUNK)
    u_rows, v_rows = _sc_gather(table128, left3, right3)
    return _tc_dist(u_rows, v_rows, li >> 18, ri >> 18)


# speedup vs baseline: 1.1837x; 1.1837x over previous
import functools

import jax
import jax.numpy as jnp
from jax import lax
from jax.experimental import pallas as pl
from jax.experimental.pallas import tpu as pltpu
from jax.experimental.pallas import tpu_sc as plsc

VOCAB = 1000000
EMBED_DIMS = 32
BATCH = 16384
EPS = 1e-05

_GROUP = 128 // EMBED_DIMS
_SGRP = 1 << 18
_NGRP = _SGRP
_NC = 2
_NS = 16
_NW = _NC * _NS
_BPW = BATCH // _NW
_CHUNK = 128
_NCHUNK = _BPW // _CHUNK

_VWIN = 8192
_NBLK = _SGRP // _VWIN


def _tc_pack_kernel(t0_ref, t1_ref, t2_ref, t3_ref, o_ref):
    x = jnp.concatenate(
        [t0_ref[...], t1_ref[...], t2_ref[...], t3_ref[...]], axis=0)
    o_ref[...] = x.T


@jax.jit
def _tc_pack(tableT):
    last_blk = VOCAB // _VWIN
    specs = [
        pl.BlockSpec(
            (EMBED_DIMS, _VWIN),
            functools.partial(
                lambda q, i: (0, jnp.minimum(q * _NBLK + i, last_blk)), q))
        for q in range(_GROUP)
    ]
    return pl.pallas_call(
        _tc_pack_kernel,
        grid=(_NBLK,),
        in_specs=specs,
        out_specs=pl.BlockSpec((_VWIN, 128), lambda i: (i, 0)),
        out_shape=jax.ShapeDtypeStruct((_NGRP, 128), jnp.float32),
    )(tableT, tableT, tableT, tableT)


def _sc_gather_kernel(table_hbm, left_hbm, right_hbm, u_hbm, v_hbm,
                      idx_v, gidx_v, qoff_v, rows_a, rows_b, sel_v, sem):
    wid = lax.axis_index("s") * _NC + lax.axis_index("c")
    base = wid * _BPW

    for src, dst in ((left_hbm, u_hbm), (right_hbm, v_hbm)):
        pltpu.sync_copy(src.at[wid], idx_v)
        for j in range(_NCHUNK):
            for k in range(_CHUNK // 16):
                sl = pl.ds(k * 16, 16)
                raw = idx_v[j, sl]
                gidx_v[j, sl] = raw & (_SGRP - 1)
                qoff_v[j, sl] = (raw >> 18) * EMBED_DIMS
        bufs = (rows_a, rows_b)
        pend = pltpu.async_copy(table_hbm.at[gidx_v.at[0]], bufs[0], sem)
        for j in range(_NCHUNK):
            if j + 1 < _NCHUNK:
                nxt = pltpu.async_copy(
                    table_hbm.at[gidx_v.at[j + 1]], bufs[(j + 1) & 1], sem)
            pend.wait()
            buf = bufs[j & 1]

            def group_body(g, _, j=j, buf=buf):
                k16 = g * 16
                offs = qoff_v[j, pl.ds(k16, 16)]
                for l in range(16):
                    r = k16 + l
                    off = offs[l]
                    sel_v[j * _CHUNK + r, pl.ds(0, 16)] = buf[r, pl.ds(off, 16)]
                    sel_v[j * _CHUNK + r, pl.ds(16, 16)] = (
                        buf[r, pl.ds(off + 16, 16)])
                return ()

            lax.fori_loop(0, _CHUNK // 16, group_body, (), unroll=False)
            if j + 1 < _NCHUNK:
                pend = nxt
        pltpu.sync_copy(sel_v, dst.at[pl.ds(base, _BPW)])


@jax.jit
def _sc_gather(table128, left3, right3):
    mesh = plsc.VectorSubcoreMesh(core_axis_name="c", subcore_axis_name="s")
    kfn = functools.partial(
        pl.kernel,
        mesh=mesh,
        out_type=[jax.ShapeDtypeStruct((BATCH, EMBED_DIMS), jnp.float32),
                  jax.ShapeDtypeStruct((BATCH, EMBED_DIMS), jnp.float32)],
        scratch_types=[
            pltpu.VMEM((_NCHUNK, _CHUNK), jnp.int32),
            pltpu.VMEM((_NCHUNK, _CHUNK), jnp.int32),
            pltpu.VMEM((_NCHUNK, _CHUNK), jnp.int32),
            pltpu.VMEM((_CHUNK, 128), jnp.float32),
            pltpu.VMEM((_CHUNK, 128), jnp.float32),
            pltpu.VMEM((_BPW, EMBED_DIMS), jnp.float32),
            pltpu.SemaphoreType.DMA,
        ],
    )(_sc_gather_kernel)
    return kfn(table128, left3, right3)


_TCB = 4096


def _tc_dist_kernel(u_ref, v_ref, o_ref):
    u = u_ref[...]
    v = v_ref[...]
    uu = jnp.sum(u * u, axis=-1)
    vv = jnp.sum(v * v, axis=-1)
    uv = jnp.sum(u * v, axis=-1)
    alpha = 1.0 - uu
    alpha = jnp.where(alpha <= 0.0, EPS, alpha)
    beta = 1.0 - vv
    beta = jnp.where(beta <= 0.0, EPS, beta)
    gamma = 1.0 + 2.0 * (uu - 2.0 * uv + vv) / alpha / beta
    gamma = jnp.where(gamma < 1.0, 1.0, gamma)
    o_ref[...] = jnp.log(gamma + jnp.sqrt(gamma * gamma - 1.0))


@jax.jit
def _tc_dist(u_rows, v_rows):
    nblk = BATCH // _TCB
    return pl.pallas_call(
        _tc_dist_kernel,
        grid=(nblk,),
        in_specs=[
            pl.BlockSpec((_TCB, EMBED_DIMS), lambda i: (i, 0)),
            pl.BlockSpec((_TCB, EMBED_DIMS), lambda i: (i, 0)),
        ],
        out_specs=pl.BlockSpec((_TCB,), lambda i: (i,)),
        out_shape=jax.ShapeDtypeStruct((BATCH,), jnp.float32),
    )(u_rows, v_rows)


def kernel(left_idx, right_idx, table):
    li = left_idx.astype(jnp.int32)
    ri = right_idx.astype(jnp.int32)
    table128 = _tc_pack(jnp.swapaxes(table, 0, 1))
    left3 = li.reshape(_NW, _NCHUNK, _CHUNK)
    right3 = ri.reshape(_NW, _NCHUNK, _CHUNK)
    u_rows, v_rows = _sc_gather(table128, left3, right3)
    return _tc_dist(u_rows, v_rows)


# speedup vs baseline: 1.1977x; 1.0118x over previous
import functools

import jax
import jax.numpy as jnp
from jax import lax
from jax.experimental import pallas as pl
from jax.experimental.pallas import tpu as pltpu
from jax.experimental.pallas import tpu_sc as plsc

VOCAB = 1000000
EMBED_DIMS = 32
BATCH = 16384
EPS = 1e-05

_GROUP = 128 // EMBED_DIMS
_SGRP = 1 << 18
_NGRP = _SGRP
_NC = 2
_NS = 16
_NW = _NC * _NS
_BPW = BATCH // _NW
_CHUNK = 128
_NCHUNK = _BPW // _CHUNK

_VWIN = 16384
_NBLK = _SGRP // _VWIN


def _tc_pack_kernel(t0_ref, t1_ref, t2_ref, t3_ref, o_ref):
    x = jnp.concatenate(
        [t0_ref[...], t1_ref[...], t2_ref[...], t3_ref[...]], axis=0)
    o_ref[...] = x.T


@jax.jit
def _tc_pack(tableT):
    last_blk = VOCAB // _VWIN
    specs = [
        pl.BlockSpec(
            (EMBED_DIMS, _VWIN),
            functools.partial(
                lambda q, i: (0, jnp.minimum(q * _NBLK + i, last_blk)), q))
        for q in range(_GROUP)
    ]
    return pl.pallas_call(
        _tc_pack_kernel,
        grid=(_NBLK,),
        in_specs=specs,
        out_specs=pl.BlockSpec((_VWIN, 128), lambda i: (i, 0)),
        out_shape=jax.ShapeDtypeStruct((_NGRP, 128), jnp.float32),
    )(tableT, tableT, tableT, tableT)


def _sc_gather_kernel(table_hbm, left_hbm, right_hbm, u_hbm, v_hbm,
                      idx_v, gidx_v, qoff_v, rows_a, rows_b, sel_v, sem):
    wid = lax.axis_index("s") * _NC + lax.axis_index("c")
    base = wid * _BPW

    for src, dst in ((left_hbm, u_hbm), (right_hbm, v_hbm)):
        pltpu.sync_copy(src.at[wid], idx_v)
        for j in range(_NCHUNK):
            for k in range(_CHUNK // 16):
                sl = pl.ds(k * 16, 16)
                raw = idx_v[j, sl]
                gidx_v[j, sl] = raw & (_SGRP - 1)
                qoff_v[j, sl] = (raw >> 18) * EMBED_DIMS
        bufs = (rows_a, rows_b)
        pend = pltpu.async_copy(table_hbm.at[gidx_v.at[0]], bufs[0], sem)
        for j in range(_NCHUNK):
            if j + 1 < _NCHUNK:
                nxt = pltpu.async_copy(
                    table_hbm.at[gidx_v.at[j + 1]], bufs[(j + 1) & 1], sem)
            pend.wait()
            buf = bufs[j & 1]

            def group_body(g, _, j=j, buf=buf):
                k16 = g * 16
                offs = qoff_v[j, pl.ds(k16, 16)]
                for l in range(16):
                    r = k16 + l
                    off = offs[l]
                    sel_v[j * _CHUNK + r, pl.ds(0, 16)] = buf[r, pl.ds(off, 16)]
                    sel_v[j * _CHUNK + r, pl.ds(16, 16)] = (
                        buf[r, pl.ds(off + 16, 16)])
                return ()

            lax.fori_loop(0, _CHUNK // 16, group_body, (), unroll=False)
            if j + 1 < _NCHUNK:
                pend = nxt
        pltpu.sync_copy(sel_v, dst.at[pl.ds(base, _BPW)])


@jax.jit
def _sc_gather(table128, left3, right3):
    mesh = plsc.VectorSubcoreMesh(core_axis_name="c", subcore_axis_name="s")
    kfn = functools.partial(
        pl.kernel,
        mesh=mesh,
        out_type=[jax.ShapeDtypeStruct((BATCH, EMBED_DIMS), jnp.float32),
                  jax.ShapeDtypeStruct((BATCH, EMBED_DIMS), jnp.float32)],
        scratch_types=[
            pltpu.VMEM((_NCHUNK, _CHUNK), jnp.int32),
            pltpu.VMEM((_NCHUNK, _CHUNK), jnp.int32),
            pltpu.VMEM((_NCHUNK, _CHUNK), jnp.int32),
            pltpu.VMEM((_CHUNK, 128), jnp.float32),
            pltpu.VMEM((_CHUNK, 128), jnp.float32),
            pltpu.VMEM((_BPW, EMBED_DIMS), jnp.float32),
            pltpu.SemaphoreType.DMA,
        ],
    )(_sc_gather_kernel)
    return kfn(table128, left3, right3)


_TCB = 8192


def _tc_dist_kernel(u_ref, v_ref, o_ref):
    u = u_ref[...]
    v = v_ref[...]
    uu = jnp.sum(u * u, axis=-1)
    vv = jnp.sum(v * v, axis=-1)
    uv = jnp.sum(u * v, axis=-1)
    alpha = 1.0 - uu
    alpha = jnp.where(alpha <= 0.0, EPS, alpha)
    beta = 1.0 - vv
    beta = jnp.where(beta <= 0.0, EPS, beta)
    gamma = 1.0 + 2.0 * (uu - 2.0 * uv + vv) / alpha / beta
    gamma = jnp.where(gamma < 1.0, 1.0, gamma)
    o_ref[...] = jnp.log(gamma + jnp.sqrt(gamma * gamma - 1.0))


@jax.jit
def _tc_dist(u_rows, v_rows):
    nblk = BATCH // _TCB
    return pl.pallas_call(
        _tc_dist_kernel,
        grid=(nblk,),
        in_specs=[
            pl.BlockSpec((_TCB, EMBED_DIMS), lambda i: (i, 0)),
            pl.BlockSpec((_TCB, EMBED_DIMS), lambda i: (i, 0)),
        ],
        out_specs=pl.BlockSpec((_TCB,), lambda i: (i,)),
        out_shape=jax.ShapeDtypeStruct((BATCH,), jnp.float32),
    )(u_rows, v_rows)


def kernel(left_idx, right_idx, table):
    li = left_idx.astype(jnp.int32)
    ri = right_idx.astype(jnp.int32)
    table128 = _tc_pack(jnp.swapaxes(table, 0, 1))
    left3 = li.reshape(_NW, _NCHUNK, _CHUNK)
    right3 = ri.reshape(_NW, _NCHUNK, _CHUNK)
    u_rows, v_rows = _sc_gather(table128, left3, right3)
    return _tc_dist(u_rows, v_rows)
